# baseline (speedup 1.0000x reference)
import jax
import jax.numpy as jnp
from jax.experimental import pallas as pl
from jax.experimental.pallas import tpu as pltpu

_P = 4096
_A = 8
_H = 64
_R = 1024
_G1 = 0.99
_G2 = 0.99 * 0.99


def _fused_kernel(obs_ref, W_enc_ref, b_enc_ref, W_trans_ref, b_trans_ref,
                  W_t2g_ref, b_t2g_ref, W_edge_ref, b_edge_ref,
                  W_msg_ref, b_msg_ref, W_upd_ref, b_upd_ref,
                  W_actor_ref, b_actor_ref, W_critic_ref, b_critic_ref,
                  value_ref, action_ref, logp_ref):
    f32 = jnp.float32

    def dot(a, b):
        return jnp.dot(a, b, preferred_element_type=f32)

    relu = jax.nn.relu
    cat = jnp.concatenate

    obs = obs_ref[...]
    W_enc = W_enc_ref[...]
    b_enc = b_enc_ref[...]
    W_trans = W_trans_ref[...]
    b_trans = b_trans_ref[...]
    W_t2g = W_t2g_ref[...]
    b_t2g = b_t2g_ref[...]
    W_edge = W_edge_ref[...]
    b_edge = b_edge_ref[...]
    W_msg = W_msg_ref[...]
    b_msg = b_msg_ref[...]
    W_upd = W_upd_ref[...]
    b_upd = b_upd_ref[...]

    def onehot(rows, a):
        return (jax.lax.broadcasted_iota(jnp.int32, (rows, _A), 1)
                == a).astype(f32)

    eye8 = (jax.lax.broadcasted_iota(jnp.int32, (_A, _A), 0)
            == jax.lax.broadcasted_iota(jnp.int32, (_A, _A), 1)).astype(f32)
    E1 = dot(cat([eye8, jnp.full((_A, 1), _G1, f32)], 1), W_edge) + b_edge
    E2 = dot(cat([eye8, jnp.full((_A, 1), _G2, f32)], 1), W_edge) + b_edge

    L = jnp.tanh(dot(obs, W_enc) + b_enc)
    ch1 = cat([L + jnp.tanh(dot(cat([L, onehot(_R, a)], 1), W_trans)
                            + b_trans)
               for a in range(_A)], axis=0)

    h_root = dot(L, W_t2g) + b_t2g
    h1 = dot(ch1, W_t2g) + b_t2g

    n1 = _A * _R
    agg_l1 = None
    for b in range(_A):
        leaf = ch1 + jnp.tanh(dot(cat([ch1, onehot(n1, b)], 1), W_trans)
                              + b_trans)
        h_leaf = dot(leaf, W_t2g) + b_t2g
        e_b = jnp.broadcast_to(E2[b:b + 1, :], (n1, _H))
        msg = relu(dot(cat([h_leaf, h1, e_b], 1), W_msg) + b_msg)
        agg_l1 = msg if agg_l1 is None else agg_l1 + msg
    out1_l1 = relu(dot(cat([h1, agg_l1], 1), W_upd) + b_upd)

    def root_msgs(h_send, h_recv):
        agg = None
        for a in range(_A):
            e_a = jnp.broadcast_to(E1[a:a + 1, :], (_R, _H))
            msg = relu(dot(cat([h_send[a * _R:(a + 1) * _R, :], h_recv, e_a],
                               1), W_msg) + b_msg)
            agg = msg if agg is None else agg + msg
        return agg

    agg1 = root_msgs(h1, h_root)
    out1_root = relu(dot(cat([h_root, agg1], 1), W_upd) + b_upd)

    h2_root = out1_root + h_root
    h2_l1 = out1_l1 + h1
    agg2 = root_msgs(h2_l1, h2_root)
    out2 = relu(dot(cat([h2_root, agg2], 1), W_upd) + b_upd)

    feat = cat([L, out2], axis=1)
    policy = dot(feat, W_actor_ref[...]) + b_actor_ref[...]
    value_ref[...] = dot(feat, W_critic_ref[...]) + b_critic_ref[...]
    action_ref[...] = jnp.argmax(policy, axis=-1).astype(jnp.int32)
    pmax = jnp.max(policy, axis=-1, keepdims=True)
    logp_ref[...] = -jnp.log(jnp.sum(jnp.exp(policy - pmax), axis=-1,
                                     keepdims=True))


def kernel(observations, W_enc, b_enc, W_trans, b_trans, W_t2g, b_t2g,
           W_edge, b_edge, W_msg, b_msg, W_upd, b_upd,
           W_actor, b_actor, W_critic, b_critic):
    n_tiles = _P // _R
    row2 = lambda x: x.reshape(1, -1)

    def full(arr):
        return pl.BlockSpec(arr.shape, lambda i: (0,) * arr.ndim)

    args = (
        observations,
        W_enc, row2(b_enc),
        W_trans, row2(b_trans),
        W_t2g, row2(b_t2g),
        W_edge, row2(b_edge),
        W_msg, row2(b_msg),
        W_upd, row2(b_upd),
        W_actor, row2(b_actor),
        W_critic, row2(b_critic),
    )
    in_specs = [pl.BlockSpec((_R, _H), lambda i: (i, 0))]
    in_specs += [full(a) for a in args[1:]]
    out_specs = [
        pl.BlockSpec((_R, 1), lambda i: (i, 0)),
        pl.BlockSpec((_R,), lambda i: (i,)),
        pl.BlockSpec((_R, 1), lambda i: (i, 0)),
    ]
    out_shapes = [
        jax.ShapeDtypeStruct((_P, 1), jnp.float32),
        jax.ShapeDtypeStruct((_P,), jnp.int32),
        jax.ShapeDtypeStruct((_P, 1), jnp.float32),
    ]
    value, action, logp = pl.pallas_call(
        _fused_kernel,
        grid=(n_tiles,),
        in_specs=in_specs,
        out_specs=out_specs,
        out_shape=out_shapes,
        compiler_params=pltpu.CompilerParams(
            dimension_semantics=("parallel",),
            vmem_limit_bytes=100 * 1024 * 1024),
    )(*args)
    return (value, action, logp)


# speedup vs baseline: 1.0473x; 1.0473x over previous
import jax
import jax.numpy as jnp
from jax.experimental import pallas as pl
from jax.experimental.pallas import tpu as pltpu

_P = 4096
_A = 8
_H = 64
_R = 1024
_G1 = 0.99
_G2 = 0.99 * 0.99


def _fused_kernel(obs_ref, W_enc_ref, b_enc_ref, W_trans_ref, b_trans_ref,
                  W_t2g_ref, b_t2g_ref, W_edge_ref, b_edge_ref,
                  W_msg_ref, b_msg_ref, W_upd_ref, b_upd_ref,
                  W_actor_ref, b_actor_ref, W_critic_ref, b_critic_ref,
                  value_ref, action_ref, logp_ref):
    f32 = jnp.float32

    def dot(a, b):
        return jnp.dot(a, b, preferred_element_type=f32)

    relu = jax.nn.relu
    cat = jnp.concatenate

    obs = obs_ref[...]
    W_enc = W_enc_ref[...]
    b_enc = b_enc_ref[...]
    W_trans = W_trans_ref[...]
    b_trans = b_trans_ref[...]
    W_t2g = W_t2g_ref[...]
    b_t2g = b_t2g_ref[...]
    W_edge = W_edge_ref[...]
    b_edge = b_edge_ref[...]
    W_msg = W_msg_ref[...]
    b_msg = b_msg_ref[...]
    W_upd = W_upd_ref[...]
    b_upd = b_upd_ref[...]

    def onehot(rows, a):
        return (jax.lax.broadcasted_iota(jnp.int32, (rows, _A), 1)
                == a).astype(f32)

    eye8 = (jax.lax.broadcasted_iota(jnp.int32, (_A, _A), 0)
            == jax.lax.broadcasted_iota(jnp.int32, (_A, _A), 1)).astype(f32)
    E1 = dot(cat([eye8, jnp.full((_A, 1), _G1, f32)], 1), W_edge) + b_edge
    E2 = dot(cat([eye8, jnp.full((_A, 1), _G2, f32)], 1), W_edge) + b_edge

    L = jnp.tanh(dot(obs, W_enc) + b_enc)
    ch1 = cat([L + jnp.tanh(dot(cat([L, onehot(_R, a)], 1), W_trans)
                            + b_trans)
               for a in range(_A)], axis=0)

    h_root = dot(L, W_t2g) + b_t2g
    h1 = dot(ch1, W_t2g) + b_t2g

    n1 = _A * _R
    agg_l1 = None
    for b in range(_A):
        leaf = ch1 + jnp.tanh(dot(cat([ch1, onehot(n1, b)], 1), W_trans)
                              + b_trans)
        h_leaf = dot(leaf, W_t2g) + b_t2g
        e_b = jnp.broadcast_to(E2[b:b + 1, :], (n1, _H))
        msg = relu(dot(cat([h_leaf, h1, e_b], 1), W_msg) + b_msg)
        agg_l1 = msg if agg_l1 is None else agg_l1 + msg
    out1_l1 = relu(dot(cat([h1, agg_l1], 1), W_upd) + b_upd)

    def root_msgs(h_send, h_recv):
        agg = None
        for a in range(_A):
            e_a = jnp.broadcast_to(E1[a:a + 1, :], (_R, _H))
            msg = relu(dot(cat([h_send[a * _R:(a + 1) * _R, :], h_recv, e_a],
                               1), W_msg) + b_msg)
            agg = msg if agg is None else agg + msg
        return agg

    agg1 = root_msgs(h1, h_root)
    out1_root = relu(dot(cat([h_root, agg1], 1), W_upd) + b_upd)

    h2_root = out1_root + h_root
    h2_l1 = out1_l1 + h1
    agg2 = root_msgs(h2_l1, h2_root)
    out2 = relu(dot(cat([h2_root, agg2], 1), W_upd) + b_upd)

    feat = cat([L, out2], axis=1)
    policy = dot(feat, W_actor_ref[...]) + b_actor_ref[...]
    value_ref[...] = dot(feat, W_critic_ref[...]) + b_critic_ref[...]
    action_ref[...] = jnp.argmax(policy, axis=-1).astype(jnp.int32)[:, None]
    pmax = jnp.max(policy, axis=-1, keepdims=True)
    logp_ref[...] = -jnp.log(jnp.sum(jnp.exp(policy - pmax), axis=-1,
                                     keepdims=True))


def kernel(observations, W_enc, b_enc, W_trans, b_trans, W_t2g, b_t2g,
           W_edge, b_edge, W_msg, b_msg, W_upd, b_upd,
           W_actor, b_actor, W_critic, b_critic):
    n_tiles = _P // _R
    row2 = lambda x: x.reshape(1, -1)

    def full(arr):
        return pl.BlockSpec(arr.shape, lambda i: (0,) * arr.ndim)

    args = (
        observations,
        W_enc, row2(b_enc),
        W_trans, row2(b_trans),
        W_t2g, row2(b_t2g),
        W_edge, row2(b_edge),
        W_msg, row2(b_msg),
        W_upd, row2(b_upd),
        W_actor, row2(b_actor),
        W_critic, row2(b_critic),
    )
    in_specs = [pl.BlockSpec((_R, _H), lambda i: (i, 0))]
    in_specs += [full(a) for a in args[1:]]
    out_specs = [
        pl.BlockSpec((_R, 1), lambda i: (i, 0)),
        pl.BlockSpec((_R, 1), lambda i: (i, 0)),
        pl.BlockSpec((_R, 1), lambda i: (i, 0)),
    ]
    out_shapes = [
        jax.ShapeDtypeStruct((_P, 1), jnp.float32),
        jax.ShapeDtypeStruct((_P, 1), jnp.int32),
        jax.ShapeDtypeStruct((_P, 1), jnp.float32),
    ]
    value, action, logp = pl.pallas_call(
        _fused_kernel,
        grid=(n_tiles,),
        in_specs=in_specs,
        out_specs=out_specs,
        out_shape=out_shapes,
        compiler_params=pltpu.CompilerParams(
            dimension_semantics=("parallel",),
            vmem_limit_bytes=100 * 1024 * 1024),
    )(*args)
    return (value, action[:, 0], logp)
